# baseline (speedup 1.0000x reference)
import functools

import jax
import jax.numpy as jnp
from jax import lax
from jax.experimental import pallas as pl
from jax.experimental.pallas import tpu as pltpu
from jax.experimental.pallas import tpu_sc as plsc

VOCAB = 49408
MAX_POS = 77
EMBED = 512
HALF_D = EMBED // 2
BATCH = 1024
TOTAL = BATCH * MAX_POS

NUM_CORES = 2
NUM_SUBCORES = 16
NUM_WORKERS = NUM_CORES * NUM_SUBCORES
CHUNK = 112
ADD_BLK_B = 32
ADD_BLK = ADD_BLK_B * MAX_POS

PACK_BLK = 1544

N_PHASES = 2
PHASE_B = BATCH // N_PHASES
PHASE_ROWS = PHASE_B * MAX_POS


def _pack_body(x_ref, o_ref):
    xu = lax.bitcast_convert_type(x_ref[...], jnp.uint32)
    lo = xu[:, :HALF_D]
    hi = xu[:, HALF_D:]
    o_ref[...] = lax.bitcast_convert_type(
        (hi & jnp.uint32(0xFFFF0000)) | (lo >> 16), jnp.int32
    )


def _pack_table(table):
    return pl.pallas_call(
        _pack_body,
        grid=(VOCAB // PACK_BLK,),
        in_specs=[pl.BlockSpec((PACK_BLK, EMBED), lambda i: (i, 0))],
        out_specs=pl.BlockSpec((PACK_BLK, HALF_D), lambda i: (i, 0)),
        out_shape=jax.ShapeDtypeStruct((VOCAB, HALF_D), jnp.int32),
        compiler_params=pltpu.CompilerParams(dimension_semantics=("parallel",)),
    )(table)


def _make_gather_kernel(total_rows, width, dtype):
    b_per_w = total_rows // NUM_WORKERS
    n_chunks = b_per_w // CHUNK
    assert b_per_w % CHUNK == 0
    mesh = plsc.VectorSubcoreMesh(core_axis_name="c", subcore_axis_name="s")

    @functools.partial(
        pl.kernel,
        mesh=mesh,
        out_type=jax.ShapeDtypeStruct((total_rows, width), dtype),
        scratch_types=[
            pltpu.VMEM((b_per_w,), jnp.int32),
            pltpu.VMEM((CHUNK, width), dtype),
            pltpu.VMEM((CHUNK, width), dtype),
            pltpu.SemaphoreType.DMA,
            pltpu.SemaphoreType.DMA,
        ],
    )
    def gather_kernel(table_hbm, idx_hbm, out_hbm, idx_v, rows0, rows1, sem0, sem1):
        wid = lax.axis_index("s") * NUM_CORES + lax.axis_index("c")
        base = wid * b_per_w
        pltpu.sync_copy(idx_hbm.at[pl.ds(base, b_per_w)], idx_v)
        bufs = (rows0, rows1)
        sems = (sem0, sem1)

        for k in range(min(2, n_chunks)):
            off = k * CHUNK
            pltpu.async_copy(
                table_hbm.at[idx_v.at[pl.ds(off, CHUNK)]], bufs[k], sems[k]
            )

        @pl.loop(0, n_chunks, step=2)
        def _(g):
            for k in range(2):
                c = g + k

                @pl.when(c < n_chunks)
                def _():
                    off = pl.multiple_of(c * CHUNK, CHUNK)
                    pltpu.make_async_copy(
                        table_hbm.at[idx_v.at[pl.ds(off, CHUNK)]], bufs[k], sems[k]
                    ).wait()
                    pltpu.sync_copy(bufs[k], out_hbm.at[pl.ds(base + off, CHUNK)])

                    @pl.when(c + 2 < n_chunks)
                    def _():
                        noff = pl.multiple_of((c + 2) * CHUNK, CHUNK)
                        pltpu.async_copy(
                            table_hbm.at[idx_v.at[pl.ds(noff, CHUNK)]],
                            bufs[k],
                            sems[k],
                        )

    return gather_kernel


_GATHER_F32 = _make_gather_kernel(PHASE_ROWS, EMBED, jnp.float32)
_GATHER_PACKED = _make_gather_kernel(PHASE_ROWS, HALF_D, jnp.int32)


def _unpack_add(x_packed, p_ref):
    xu = lax.bitcast_convert_type(x_packed, jnp.uint32)
    lo = lax.bitcast_convert_type(xu << 16, jnp.float32)
    hi = lax.bitcast_convert_type(xu & jnp.uint32(0xFFFF0000), jnp.float32)
    vals = jnp.concatenate([lo, hi], axis=-1)
    return vals.reshape(ADD_BLK_B, MAX_POS, EMBED) + p_ref[...]


def _add_body_f32(x_ref, p_ref, o_ref):
    o_ref[...] = x_ref[...].reshape(ADD_BLK_B, MAX_POS, EMBED) + p_ref[...]


def _add_body_aliased(x_ref, p_ref, _prev_ref, o_ref):
    o_ref[...] = _unpack_add(x_ref[...], p_ref)


def _pos_add_phase(tok_emb, pos3, phase, prev_out):
    grid = (PHASE_ROWS // ADD_BLK,)
    blk_off = phase * (PHASE_B // ADD_BLK_B)
    in_width = EMBED if prev_out is None else HALF_D
    in_specs = [
        pl.BlockSpec((ADD_BLK, in_width), lambda i: (i, 0)),
        pl.BlockSpec((1, MAX_POS, EMBED), lambda i: (0, 0, 0)),
    ]
    out_spec = pl.BlockSpec(
        (ADD_BLK_B, MAX_POS, EMBED), lambda i: (i + blk_off, 0, 0)
    )
    out_shape = jax.ShapeDtypeStruct((BATCH, MAX_POS, EMBED), jnp.float32)
    if prev_out is None:
        return pl.pallas_call(
            _add_body_f32,
            grid=grid,
            in_specs=in_specs,
            out_specs=out_spec,
            out_shape=out_shape,
            compiler_params=pltpu.CompilerParams(dimension_semantics=("parallel",)),
        )(tok_emb, pos3)
    return pl.pallas_call(
        _add_body_aliased,
        grid=grid,
        in_specs=in_specs + [pl.BlockSpec(memory_space=pl.ANY)],
        out_specs=out_spec,
        out_shape=out_shape,
        input_output_aliases={2: 0},
        compiler_params=pltpu.CompilerParams(dimension_semantics=("parallel",)),
    )(tok_emb, pos3, prev_out)


def kernel(input_tokens, token_table, position_table):
    idx = input_tokens.reshape(TOTAL).astype(jnp.int32)
    idx0 = lax.slice(idx, (0,), (PHASE_ROWS,))
    idx1 = lax.slice(idx, (PHASE_ROWS,), (TOTAL,))
    pos3 = position_table[None]
    packed = _pack_table(token_table)
    tok0 = _GATHER_F32(token_table, idx0)
    tok1 = _GATHER_PACKED(packed, idx1)
    out = _pos_add_phase(tok0, pos3, 0, None)
    out = _pos_add_phase(tok1, pos3, 1, out)
    return out


# speedup vs baseline: 1.0944x; 1.0944x over previous
import functools

import jax
import jax.numpy as jnp
from jax import lax
from jax.experimental import pallas as pl
from jax.experimental.pallas import tpu as pltpu
from jax.experimental.pallas import tpu_sc as plsc

VOCAB = 49408
MAX_POS = 77
EMBED = 512
HALF_D = EMBED // 2
BATCH = 1024
TOTAL = BATCH * MAX_POS

NUM_CORES = 2
NUM_SUBCORES = 16
NUM_WORKERS = NUM_CORES * NUM_SUBCORES
CHUNK = 112
ADD_BLK_B = 32
ADD_BLK = ADD_BLK_B * MAX_POS

PACK_BLK = 1544

N_PHASES = 2
PHASE_B = BATCH // N_PHASES
PHASE_ROWS = PHASE_B * MAX_POS


def _pack_body(x_ref, o_ref):
    xu = lax.bitcast_convert_type(x_ref[...], jnp.uint32)
    lo = xu[:, :HALF_D]
    hi = xu[:, HALF_D:]
    o_ref[...] = lax.bitcast_convert_type(
        (hi & jnp.uint32(0xFFFF0000)) | (lo >> 16), jnp.int32
    )


def _pack_table(table):
    return pl.pallas_call(
        _pack_body,
        grid=(VOCAB // PACK_BLK,),
        in_specs=[pl.BlockSpec((PACK_BLK, EMBED), lambda i: (i, 0))],
        out_specs=pl.BlockSpec((PACK_BLK, HALF_D), lambda i: (i, 0)),
        out_shape=jax.ShapeDtypeStruct((VOCAB, HALF_D), jnp.int32),
        compiler_params=pltpu.CompilerParams(dimension_semantics=("parallel",)),
    )(table)


def _make_gather_kernel(total_rows):
    b_per_w = total_rows // NUM_WORKERS
    n_chunks = b_per_w // CHUNK
    assert b_per_w % CHUNK == 0
    mesh = plsc.VectorSubcoreMesh(core_axis_name="c", subcore_axis_name="s")

    @functools.partial(
        pl.kernel,
        mesh=mesh,
        out_type=jax.ShapeDtypeStruct((total_rows, HALF_D), jnp.int32),
        scratch_types=[
            pltpu.VMEM((b_per_w,), jnp.int32),
            pltpu.VMEM((CHUNK, HALF_D), jnp.int32),
            pltpu.VMEM((CHUNK, HALF_D), jnp.int32),
            pltpu.SemaphoreType.DMA,
            pltpu.SemaphoreType.DMA,
        ],
    )
    def gather_kernel(table_hbm, idx_hbm, out_hbm, idx_v, rows0, rows1, sem0, sem1):
        wid = lax.axis_index("s") * NUM_CORES + lax.axis_index("c")
        base = wid * b_per_w
        pltpu.sync_copy(idx_hbm.at[pl.ds(base, b_per_w)], idx_v)
        bufs = (rows0, rows1)
        sems = (sem0, sem1)

        for k in range(min(2, n_chunks)):
            off = k * CHUNK
            pltpu.async_copy(
                table_hbm.at[idx_v.at[pl.ds(off, CHUNK)]], bufs[k], sems[k]
            )

        @pl.loop(0, n_chunks, step=2)
        def _(g):
            for k in range(2):
                c = g + k

                @pl.when(c < n_chunks)
                def _():
                    off = pl.multiple_of(c * CHUNK, CHUNK)
                    pltpu.make_async_copy(
                        table_hbm.at[idx_v.at[pl.ds(off, CHUNK)]], bufs[k], sems[k]
                    ).wait()
                    pltpu.sync_copy(bufs[k], out_hbm.at[pl.ds(base + off, CHUNK)])

                    @pl.when(c + 2 < n_chunks)
                    def _():
                        noff = pl.multiple_of((c + 2) * CHUNK, CHUNK)
                        pltpu.async_copy(
                            table_hbm.at[idx_v.at[pl.ds(noff, CHUNK)]],
                            bufs[k],
                            sems[k],
                        )

    return gather_kernel


_GATHER_PHASE = _make_gather_kernel(PHASE_ROWS)


def _unpack_add(x_packed, p_ref):
    xu = lax.bitcast_convert_type(x_packed, jnp.uint32)
    lo = lax.bitcast_convert_type(xu << 16, jnp.float32)
    hi = lax.bitcast_convert_type(xu & jnp.uint32(0xFFFF0000), jnp.float32)
    vals = jnp.concatenate([lo, hi], axis=-1)
    return vals.reshape(ADD_BLK_B, MAX_POS, EMBED) + p_ref[...]


def _add_body(x_ref, p_ref, o_ref):
    o_ref[...] = _unpack_add(x_ref[...], p_ref)


def _add_body_aliased(x_ref, p_ref, _prev_ref, o_ref):
    o_ref[...] = _unpack_add(x_ref[...], p_ref)


def _pos_add_phase(tok_emb, pos3, phase, prev_out):
    grid = (PHASE_ROWS // ADD_BLK,)
    blk_off = phase * (PHASE_B // ADD_BLK_B)
    in_specs = [
        pl.BlockSpec((ADD_BLK, HALF_D), lambda i: (i, 0)),
        pl.BlockSpec((1, MAX_POS, EMBED), lambda i: (0, 0, 0)),
    ]
    out_spec = pl.BlockSpec(
        (ADD_BLK_B, MAX_POS, EMBED), lambda i: (i + blk_off, 0, 0)
    )
    out_shape = jax.ShapeDtypeStruct((BATCH, MAX_POS, EMBED), jnp.float32)
    if prev_out is None:
        return pl.pallas_call(
            _add_body,
            grid=grid,
            in_specs=in_specs,
            out_specs=out_spec,
            out_shape=out_shape,
            compiler_params=pltpu.CompilerParams(dimension_semantics=("parallel",)),
        )(tok_emb, pos3)
    return pl.pallas_call(
        _add_body_aliased,
        grid=grid,
        in_specs=in_specs + [pl.BlockSpec(memory_space=pl.ANY)],
        out_specs=out_spec,
        out_shape=out_shape,
        input_output_aliases={2: 0},
        compiler_params=pltpu.CompilerParams(dimension_semantics=("parallel",)),
    )(tok_emb, pos3, prev_out)


def kernel(input_tokens, token_table, position_table):
    idx = input_tokens.reshape(TOTAL).astype(jnp.int32)
    packed = _pack_table(token_table)
    pos3 = position_table[None]
    out = None
    for phase in range(N_PHASES):
        idx_p = lax.slice(idx, (phase * PHASE_ROWS,), ((phase + 1) * PHASE_ROWS,))
        tok_p = _GATHER_PHASE(packed, idx_p)
        out = _pos_add_phase(tok_p, pos3, phase, out)
    return out
